# baseline (speedup 1.0000x reference)
import functools

import jax
import jax.numpy as jnp
from jax.experimental import pallas as pl
from jax.experimental.pallas import tpu as pltpu

_PREC = jax.lax.Precision.DEFAULT
_BM = 512
_LANE = 128


def _routing(logits, top_k):
    n_exp = logits.shape[-1]
    iota = jax.lax.broadcasted_iota(jnp.int32, logits.shape, 1)
    work = logits
    keep = jnp.zeros(logits.shape, dtype=jnp.bool_)
    m0 = None
    for _ in range(top_k):
        m = jnp.max(work, axis=-1, keepdims=True)
        if m0 is None:
            m0 = m
        is_m = work == m
        idx = jnp.min(jnp.where(is_m, iota, n_exp), axis=-1, keepdims=True)
        sel = iota == idx
        keep = jnp.logical_or(keep, sel)
        work = jnp.where(sel, -jnp.inf, work)
    e = jnp.where(keep, jnp.exp(logits - m0), 0.0)
    s = jnp.sum(e, axis=-1, keepdims=True)
    return e / s


def _gate_body(z_ref, v_ref, d_ref, t_ref, w1z_ref, w1v_ref, w1d_ref, w1t_ref,
               b1_ref, a_ref, lnw_ref, lnb_ref, w2_ref, b2_ref, o_ref,
               scr_ref, *, top_k, nblk):
    i = pl.program_id(0)

    o_ref[...] = _routing(scr_ref[(i + 1) % 2], top_k)

    f32 = jnp.float32
    h = jnp.dot(z_ref[...], w1z_ref[...], precision=_PREC,
                preferred_element_type=f32)
    h += jnp.dot(v_ref[...], w1v_ref[...], precision=_PREC,
                 preferred_element_type=f32)
    h += jnp.dot(d_ref[...], w1d_ref[...], precision=_PREC,
                 preferred_element_type=f32)
    h += jnp.dot(t_ref[...], w1t_ref[...], precision=_PREC,
                 preferred_element_type=f32)
    h += b1_ref[...]
    a = a_ref[0, 0]
    h = jnp.where(h >= 0, h, a * h)
    mu = jnp.mean(h, axis=-1, keepdims=True)
    c = h - mu
    var = jnp.mean(c * c, axis=-1, keepdims=True)
    h = c * jax.lax.rsqrt(var + 1e-5) * lnw_ref[...] + lnb_ref[...]
    logits = jnp.dot(h.astype(jnp.bfloat16), w2_ref[...], precision=_PREC,
                     preferred_element_type=f32) + b2_ref[...]
    scr_ref[i % 2] = logits


def kernel(z_t, v_prior, delta_mean, trust_vector, W1, b1, prelu_a, ln_w, ln_b, W2, b2):
    B, d_z = z_t.shape
    d_v = v_prior.shape[1]
    d_d = delta_mean.shape[1]
    d_t = trust_vector.shape[1]
    hidden = W1.shape[0]
    n_exp = W2.shape[0]
    top_k = min(8, n_exp)

    d_tp = ((d_t + _LANE - 1) // _LANE) * _LANE
    t_pad = jnp.pad(trust_vector, ((0, 0), (0, d_tp - d_t)))

    bf16 = jnp.bfloat16
    zb, vb, db, tb = (x.astype(bf16) for x in
                      (z_t, v_prior, delta_mean, t_pad))

    w1t_full = W1.T.astype(bf16)
    w1z = w1t_full[:d_z]
    w1v = w1t_full[d_z:d_z + d_v]
    w1d = w1t_full[d_z + d_v:d_z + d_v + d_d]
    w1t = jnp.pad(w1t_full[d_z + d_v + d_d:], ((0, d_tp - d_t), (0, 0)))
    w2t = W2.T.astype(bf16)

    b1r = b1.reshape(1, hidden)
    lnwr = ln_w.reshape(1, hidden)
    lnbr = ln_b.reshape(1, hidden)
    b2r = b2.reshape(1, n_exp)
    ar = jnp.asarray(prelu_a, jnp.float32).reshape(1, 1)

    bm = min(_BM, B)
    nblk = B // bm
    grid = (nblk + 1,)

    def row_blk(w):
        return pl.BlockSpec((bm, w), lambda i: (jnp.minimum(i, nblk - 1), 0))

    def full_blk(r, c):
        return pl.BlockSpec((r, c), lambda i: (0, 0))

    body = functools.partial(_gate_body, top_k=top_k, nblk=nblk)
    return pl.pallas_call(
        body,
        grid=grid,
        in_specs=[
            row_blk(d_z), row_blk(d_v), row_blk(d_d), row_blk(d_tp),
            full_blk(d_z, hidden), full_blk(d_v, hidden), full_blk(d_d, hidden),
            full_blk(d_tp, hidden),
            full_blk(1, hidden), full_blk(1, 1), full_blk(1, hidden),
            full_blk(1, hidden), full_blk(hidden, n_exp), full_blk(1, n_exp),
        ],
        out_specs=pl.BlockSpec((bm, n_exp), lambda i: (jnp.maximum(i - 1, 0), 0)),
        out_shape=jax.ShapeDtypeStruct((B, n_exp), jnp.float32),
        scratch_shapes=[pltpu.VMEM((2, bm, n_exp), jnp.float32)],
    )(zb, vb, db, tb, w1z, w1v, w1d, w1t,
      b1r, ar, lnwr, lnbr, w2t, b2r)


# speedup vs baseline: 1.2209x; 1.2209x over previous
import functools

import jax
import jax.numpy as jnp
from jax.experimental import pallas as pl
from jax.experimental.pallas import tpu as pltpu

_PREC = jax.lax.Precision.DEFAULT
_BM = 512
_LANE = 128


def _routing(logits, top_k):
    n_exp = logits.shape[-1]
    iota = jax.lax.broadcasted_iota(jnp.int32, logits.shape, 1)
    work = logits
    keep = jnp.zeros(logits.shape, dtype=jnp.bool_)
    m0 = None
    for _ in range(top_k):
        m = jnp.max(work, axis=-1, keepdims=True)
        if m0 is None:
            m0 = m
        is_m = work == m
        idx = jnp.min(jnp.where(is_m, iota, n_exp), axis=-1, keepdims=True)
        sel = iota == idx
        keep = jnp.logical_or(keep, sel)
        work = jnp.where(sel, -jnp.inf, work)
    e = jnp.where(keep, jnp.exp(logits - m0), 0.0)
    s = jnp.sum(e, axis=-1, keepdims=True)
    return e / s


def _gate_body(z_ref, v_ref, d_ref, t_ref, w1z_ref, w1v_ref, w1d_ref, w1t_ref,
               b1_ref, a_ref, lnw_ref, lnb_ref, w2_ref, b2_ref, o_ref,
               scr_ref, *, top_k, nblk):
    i = pl.program_id(0)

    o_ref[...] = _routing(scr_ref[(i + 1) % 2], top_k)

    f32 = jnp.float32
    dn = (((1,), (0,)), ((), ()))

    def mm(x, w_ref):
        return jax.lax.dot_general(x, w_ref[...], dn, precision=_PREC,
                                   preferred_element_type=f32)

    h = mm(z_ref[...], w1z_ref)
    h += mm(v_ref[...], w1v_ref)
    h += mm(d_ref[...], w1d_ref)
    h += mm(t_ref[...], w1t_ref)
    h += b1_ref[...]
    a = a_ref[0, 0]
    h = jnp.where(h >= 0, h, a * h)
    mu = jnp.mean(h, axis=-1, keepdims=True)
    c = h - mu
    var = jnp.mean(c * c, axis=-1, keepdims=True)
    h = c * jax.lax.rsqrt(var + 1e-5) * lnw_ref[...] + lnb_ref[...]
    logits = mm(h, w2_ref) + b2_ref[...]
    scr_ref[i % 2] = logits


def kernel(z_t, v_prior, delta_mean, trust_vector, W1, b1, prelu_a, ln_w, ln_b, W2, b2):
    B, d_z = z_t.shape
    d_v = v_prior.shape[1]
    d_d = delta_mean.shape[1]
    d_t = trust_vector.shape[1]
    hidden = W1.shape[0]
    n_exp = W2.shape[0]
    top_k = min(8, n_exp)

    d_tp = ((d_t + _LANE - 1) // _LANE) * _LANE
    t_pad = jnp.pad(trust_vector, ((0, 0), (0, d_tp - d_t)))

    bf16 = jnp.bfloat16
    w1t_full = W1.T.astype(bf16)
    w1z = w1t_full[:d_z]
    w1v = w1t_full[d_z:d_z + d_v]
    w1d = w1t_full[d_z + d_v:d_z + d_v + d_d]
    w1t = jnp.pad(w1t_full[d_z + d_v + d_d:], ((0, d_tp - d_t), (0, 0)))
    w2t = W2.T.astype(bf16)

    b1r = b1.reshape(1, hidden)
    lnwr = ln_w.reshape(1, hidden)
    lnbr = ln_b.reshape(1, hidden)
    b2r = b2.reshape(1, n_exp)
    ar = jnp.asarray(prelu_a, jnp.float32).reshape(1, 1)

    bm = min(_BM, B)
    nblk = B // bm
    grid = (nblk + 1,)

    def row_blk(w):
        return pl.BlockSpec((bm, w), lambda i: (jnp.minimum(i, nblk - 1), 0))

    def full_blk(r, c):
        return pl.BlockSpec((r, c), lambda i: (0, 0))

    body = functools.partial(_gate_body, top_k=top_k, nblk=nblk)
    return pl.pallas_call(
        body,
        grid=grid,
        in_specs=[
            row_blk(d_z), row_blk(d_v), row_blk(d_d), row_blk(d_tp),
            full_blk(d_z, hidden), full_blk(d_v, hidden), full_blk(d_d, hidden),
            full_blk(d_tp, hidden),
            full_blk(1, hidden), full_blk(1, 1), full_blk(1, hidden),
            full_blk(1, hidden), full_blk(hidden, n_exp), full_blk(1, n_exp),
        ],
        out_specs=pl.BlockSpec((bm, n_exp), lambda i: (jnp.maximum(i - 1, 0), 0)),
        out_shape=jax.ShapeDtypeStruct((B, n_exp), jnp.float32),
        scratch_shapes=[pltpu.VMEM((2, bm, n_exp), jnp.float32)],
    )(z_t, v_prior, delta_mean, t_pad, w1z, w1v, w1d, w1t,
      b1r, ar, lnwr, lnbr, w2t, b2r)


# speedup vs baseline: 1.3225x; 1.0833x over previous
import functools

import jax
import jax.numpy as jnp
from jax.experimental import pallas as pl
from jax.experimental.pallas import tpu as pltpu

_PREC = jax.lax.Precision.DEFAULT
_BM = 512
_LANE = 128


def _routing(logits, top_k):
    n_exp = logits.shape[-1]
    iota = jax.lax.broadcasted_iota(jnp.int32, logits.shape, 1)
    work = logits
    keep = jnp.zeros(logits.shape, dtype=jnp.bool_)
    m0 = None
    for _ in range(top_k):
        m = jnp.max(work, axis=-1, keepdims=True)
        if m0 is None:
            m0 = m
        is_m = work == m
        idx = jnp.min(jnp.where(is_m, iota, n_exp), axis=-1, keepdims=True)
        sel = iota == idx
        keep = jnp.logical_or(keep, sel)
        work = jnp.where(sel, -jnp.inf, work)
    e = jnp.where(keep, jnp.exp(logits - m0), 0.0)
    s = jnp.sum(e, axis=-1, keepdims=True)
    return e / s


def _gate_body(z_ref, v_ref, d_ref, t_ref, w1z_ref, w1v_ref, w1d_ref, w1t_ref,
               b1_ref, a_ref, lnw_ref, lnb_ref, w2_ref, b2_ref, o_ref,
               hscr_ref, lscr_ref, *, top_k, nblk):
    i = pl.program_id(0)

    o_ref[...] = _routing(lscr_ref[i % 2], top_k)

    f32 = jnp.float32
    dn = (((1,), (0,)), ((), ()))

    def mm(x, w_ref):
        return jax.lax.dot_general(x, w_ref[...], dn, precision=_PREC,
                                   preferred_element_type=f32)

    hp = hscr_ref[(i + 1) % 2]
    a = a_ref[0, 0]
    hp = jnp.where(hp >= 0, hp, a * hp)
    mu = jnp.mean(hp, axis=-1, keepdims=True)
    c = hp - mu
    var = jnp.mean(c * c, axis=-1, keepdims=True)
    hp = c * jax.lax.rsqrt(var + 1e-5) * lnw_ref[...] + lnb_ref[...]
    lscr_ref[(i + 1) % 2] = mm(hp, w2_ref) + b2_ref[...]

    h = mm(z_ref[...], w1z_ref)
    h += mm(v_ref[...], w1v_ref)
    h += mm(d_ref[...], w1d_ref)
    h += mm(t_ref[...], w1t_ref)
    hscr_ref[i % 2] = h + b1_ref[...]


def kernel(z_t, v_prior, delta_mean, trust_vector, W1, b1, prelu_a, ln_w, ln_b, W2, b2):
    B, d_z = z_t.shape
    d_v = v_prior.shape[1]
    d_d = delta_mean.shape[1]
    d_t = trust_vector.shape[1]
    hidden = W1.shape[0]
    n_exp = W2.shape[0]
    top_k = min(8, n_exp)

    d_tp = ((d_t + _LANE - 1) // _LANE) * _LANE
    t_pad = jnp.pad(trust_vector, ((0, 0), (0, d_tp - d_t)))

    bf16 = jnp.bfloat16
    w1t_full = W1.T.astype(bf16)
    w1z = w1t_full[:d_z]
    w1v = w1t_full[d_z:d_z + d_v]
    w1d = w1t_full[d_z + d_v:d_z + d_v + d_d]
    w1t = jnp.pad(w1t_full[d_z + d_v + d_d:], ((0, d_tp - d_t), (0, 0)))
    w2t = W2.T.astype(bf16)

    b1r = b1.reshape(1, hidden)
    lnwr = ln_w.reshape(1, hidden)
    lnbr = ln_b.reshape(1, hidden)
    b2r = b2.reshape(1, n_exp)
    ar = jnp.asarray(prelu_a, jnp.float32).reshape(1, 1)

    bm = min(_BM, B)
    nblk = B // bm
    grid = (nblk + 2,)

    def row_blk(w):
        return pl.BlockSpec((bm, w), lambda i: (jnp.minimum(i, nblk - 1), 0))

    def full_blk(r, c):
        return pl.BlockSpec((r, c), lambda i: (0, 0))

    body = functools.partial(_gate_body, top_k=top_k, nblk=nblk)
    return pl.pallas_call(
        body,
        grid=grid,
        in_specs=[
            row_blk(d_z), row_blk(d_v), row_blk(d_d), row_blk(d_tp),
            full_blk(d_z, hidden), full_blk(d_v, hidden), full_blk(d_d, hidden),
            full_blk(d_tp, hidden),
            full_blk(1, hidden), full_blk(1, 1), full_blk(1, hidden),
            full_blk(1, hidden), full_blk(hidden, n_exp), full_blk(1, n_exp),
        ],
        out_specs=pl.BlockSpec((bm, n_exp), lambda i: (jnp.maximum(i - 2, 0), 0)),
        out_shape=jax.ShapeDtypeStruct((B, n_exp), jnp.float32),
        scratch_shapes=[pltpu.VMEM((2, bm, hidden), jnp.float32),
                        pltpu.VMEM((2, bm, n_exp), jnp.float32)],
    )(z_t, v_prior, delta_mean, t_pad, w1z, w1v, w1d, w1t,
      b1r, ar, lnwr, lnbr, w2t, b2r)
